# baseline (speedup 1.0000x reference)
import functools

import jax
import jax.numpy as jnp
from jax.experimental import pallas as pl
from jax.experimental.pallas import tpu as pltpu
from jax.experimental.pallas import tpu_sc as plsc

B, T, V, C = 32, 256, 32, 128
N_E = 1024
HID = 256
BETA = 0.25
R = B * T
RB = 512
NRB = R // RB

NC, NS = 2, 16
NW = NC * NS
CH = 128

WAVES = (
    ((0, 0),),
    ((1, 0), (6, 0), (11, 0)),
    ((2, 1), (7, 6), (12, 11)),
    ((3, 2), (8, 7), (13, 12), (16, 12), (24, 12)),
    ((4, 3), (9, 8), (14, 13), (17, 16), (25, 24)),
    ((5, 4), (10, 9), (15, 14), (18, 17), (26, 25)),
    ((19, 18), (27, 26)),
    ((20, 19), (28, 27)),
    ((21, 20), (29, 28)),
    ((22, 21), (30, 29)),
    ((23, 22), (31, 30)),
)


def _vq_tail(h, E, idx_ref, idxf_ref, loss_ref, j, r):
    hn = jnp.sum(h * h, axis=1, keepdims=True)
    en = jnp.sum(E * E, axis=1)[None, :]
    d2 = hn - 2.0 * jnp.dot(h, E.T, preferred_element_type=jnp.float32) + en
    idx = jnp.argmin(d2, axis=1).astype(jnp.int32)
    mn = jnp.min(d2, axis=1)
    idx_ref[0, 0] = idx
    idxf_ref[0, 0] = idx + j * N_E
    part = jnp.sum(mn.reshape(-1, C), axis=0)[None, None]

    @pl.when(r == 0)
    def _():
        loss_ref[...] = part

    @pl.when(r != 0)
    def _():
        loss_ref[...] += part


def _root_body(zj_ref, E_ref, idx_ref, idxf_ref, loss_ref):
    j, r = pl.program_id(0), pl.program_id(1)
    _vq_tail(zj_ref[0], E_ref[0], idx_ref, idxf_ref, loss_ref, j, r)


def _wave_body(p_ref, zj_ref, E_ref, W1a_ref, W1b_ref, b1_ref, g_ref,
               bl_ref, W2_ref, b2_ref, idx_ref, idxf_ref, loss_ref):
    j, r = pl.program_id(0), pl.program_id(1)
    h1 = (jnp.dot(p_ref[0], W1a_ref[...], preferred_element_type=jnp.float32)
          + jnp.dot(zj_ref[0], W1b_ref[...], preferred_element_type=jnp.float32)
          + b1_ref[...])
    m = jnp.mean(h1, axis=-1, keepdims=True)
    v = jnp.mean((h1 - m) ** 2, axis=-1, keepdims=True)
    h1 = (h1 - m) / jnp.sqrt(v + 1e-5) * g_ref[...] + bl_ref[...]
    h1 = jnp.maximum(h1, 0.0)
    h = jnp.dot(h1, W2_ref[...], preferred_element_type=jnp.float32) + b2_ref[...]
    _vq_tail(h, E_ref[0], idx_ref, idxf_ref, loss_ref, j, r)


def _out_specs(nj):
    out_shape = (
        jax.ShapeDtypeStruct((nj * NRB, 1, RB), jnp.int32),
        jax.ShapeDtypeStruct((nj * NRB, 1, RB), jnp.int32),
        jax.ShapeDtypeStruct((nj, 1, C), jnp.float32),
    )
    out_specs = (
        pl.BlockSpec((1, 1, RB), lambda j, r: (j * NRB + r, 0, 0)),
        pl.BlockSpec((1, 1, RB), lambda j, r: (j * NRB + r, 0, 0)),
        pl.BlockSpec((1, 1, C), lambda j, r: (j, 0, 0)),
    )
    return out_shape, out_specs


@functools.lru_cache(maxsize=None)
def _root_call(nj):
    out_shape, out_specs = _out_specs(nj)
    return pl.pallas_call(
        _root_body,
        grid=(nj, NRB),
        in_specs=[
            pl.BlockSpec((1, RB, C), lambda j, r: (j, r, 0)),
            pl.BlockSpec((1, N_E, C), lambda j, r: (j, 0, 0)),
        ],
        out_specs=out_specs,
        out_shape=out_shape,
    )


@functools.lru_cache(maxsize=None)
def _wave_call(nj):
    out_shape, out_specs = _out_specs(nj)
    full = lambda j, r: (0, 0)
    return pl.pallas_call(
        _wave_body,
        grid=(nj, NRB),
        in_specs=[
            pl.BlockSpec((1, RB, C), lambda j, r: (j, r, 0)),
            pl.BlockSpec((1, RB, C), lambda j, r: (j, r, 0)),
            pl.BlockSpec((1, N_E, C), lambda j, r: (j, 0, 0)),
            pl.BlockSpec((C, HID), full),
            pl.BlockSpec((C, HID), full),
            pl.BlockSpec((1, HID), full),
            pl.BlockSpec((1, HID), full),
            pl.BlockSpec((1, HID), full),
            pl.BlockSpec((HID, C), full),
            pl.BlockSpec((1, C), full),
        ],
        out_specs=out_specs,
        out_shape=out_shape,
    )


NBUF = 4


@functools.lru_cache(maxsize=None)
def _gather_call(nrt):
    npw = nrt // NW
    nch = npw // CH
    mesh = plsc.VectorSubcoreMesh(core_axis_name="c", subcore_axis_name="s",
                                  num_cores=NC, num_subcores=NS)

    def body(tab_ref, idx_ref, out_ref, idx_all, rows, gsem, ssem):
        wid = jax.lax.axis_index("s") * NC + jax.lax.axis_index("c")
        base = wid * npw
        pltpu.sync_copy(idx_ref.at[wid], idx_all)
        gs = [None] * nch
        ss = [None] * nch

        def store(c):
            b = c % NBUF
            ss[c] = pltpu.async_copy(
                rows.at[b], out_ref.at[pl.ds(base + c * CH, CH)], ssem.at[b])

        for c in range(nch):
            b = c % NBUF
            if c >= NBUF:
                ss[c - NBUF].wait()
            gs[c] = pltpu.async_copy(tab_ref.at[idx_all.at[c]], rows.at[b],
                                     gsem.at[b])
            if c >= 1:
                gs[c - 1].wait()
                store(c - 1)
        gs[nch - 1].wait()
        store(nch - 1)
        for c in range(max(0, nch - NBUF), nch):
            ss[c].wait()

    return pl.kernel(
        body,
        out_type=jax.ShapeDtypeStruct((nrt, C), jnp.float32),
        mesh=mesh,
        scratch_types=[
            pltpu.VMEM((nch, CH), jnp.int32),
            pltpu.VMEM((NBUF, CH, C), jnp.float32),
            pltpu.SemaphoreType.DMA((NBUF,)),
            pltpu.SemaphoreType.DMA((NBUF,)),
        ],
    )


def kernel(z, emb, W1, b1, g_ln, b_ln, W2, b2):
    zt = jnp.transpose(z, (2, 0, 1, 3)).reshape(V, R, C)
    W1a, W1b = W1[:C], W1[C:]
    b1r = b1.reshape(1, HID)
    gr = g_ln.reshape(1, HID)
    blr = b_ln.reshape(1, HID)
    b2r = b2.reshape(1, C)

    e_all = [None] * V
    i_all = [None] * V
    loss_sum = jnp.float32(0.0)
    for w, wave in enumerate(WAVES):
        joints = jnp.array([j for j, _ in wave])
        nj = len(wave)
        zw = zt[joints]
        Ew = emb[joints]
        if w == 0:
            idx, idxf, lp = _root_call(nj)(zw, Ew)
        else:
            pw = jnp.stack([e_all[p] for _, p in wave])
            idx, idxf, lp = _wave_call(nj)(pw, zw, Ew, W1a, W1b, b1r, gr,
                                           blr, W2, b2r)
        e = _gather_call(nj * R)(Ew.reshape(nj * N_E, C),
                                 idxf.reshape(NW, -1, CH))
        e = e.reshape(nj, R, C)
        idx = idx.reshape(nj, R)
        for k, (j, _) in enumerate(wave):
            e_all[j] = e[k]
            i_all[j] = idx[k]
        loss_sum = loss_sum + jnp.sum(lp)

    z_q = jnp.stack(e_all, axis=0).reshape(V, B, T, C).transpose(1, 2, 0, 3)
    indices = jnp.stack(i_all, axis=0).reshape(V, B, T).transpose(1, 2, 0)
    total = (1.0 + BETA) * loss_sum / (V * R * C)
    return z_q, total, indices


# speedup vs baseline: 1.7561x; 1.7561x over previous
import functools

import jax
import jax.numpy as jnp
from jax.experimental import pallas as pl
from jax.experimental.pallas import tpu as pltpu

B, T, V, C = 32, 256, 32, 128
N_E = 1024
HID = 256
BETA = 0.25
R = B * T
RB = 512
NRB = R // RB

WAVES = (
    ((0, 0),),
    ((1, 0), (6, 0), (11, 0)),
    ((2, 1), (7, 6), (12, 11)),
    ((3, 2), (8, 7), (13, 12), (16, 12), (24, 12)),
    ((4, 3), (9, 8), (14, 13), (17, 16), (25, 24)),
    ((5, 4), (10, 9), (15, 14), (18, 17), (26, 25)),
    ((19, 18), (27, 26)),
    ((20, 19), (28, 27)),
    ((21, 20), (29, 28)),
    ((22, 21), (30, 29)),
    ((23, 22), (31, 30)),
)


def _vq_tail(h, E, e_ref, idx_ref, loss_ref, r):
    hn = jnp.sum(h * h, axis=1, keepdims=True)
    en = jnp.sum(E * E, axis=1)[None, :]
    d2 = hn - 2.0 * jnp.dot(h, E.T, preferred_element_type=jnp.float32) + en
    idx = jnp.argmin(d2, axis=1).astype(jnp.int32)
    oh = (jax.lax.broadcasted_iota(jnp.int32, (RB, N_E), 1) == idx[:, None])
    e = jnp.dot(oh.astype(jnp.float32), E, preferred_element_type=jnp.float32)
    diff = e - h
    e_ref[:, 0, 0, :] = e
    idx_ref[0, 0] = idx
    part = jnp.sum(diff * diff, axis=0, keepdims=True)[None]

    @pl.when(r == 0)
    def _():
        loss_ref[...] = part

    @pl.when(r != 0)
    def _():
        loss_ref[...] += part


def _root_body(z_ref, E_ref, e_ref, idx_ref, loss_ref):
    r = pl.program_id(0)
    _vq_tail(z_ref[0], E_ref[0], e_ref, idx_ref, loss_ref, r)


def _wave_body(jids_ref, pids_ref, ebuf_ref, z_ref, E_ref, W1a_ref, W1b_ref,
               b1_ref, g_ref, bl_ref, W2_ref, b2_ref, e_ref, idx_ref,
               loss_ref):
    r = pl.program_id(1)
    p = ebuf_ref[:, 0, 0, :]
    h1 = (jnp.dot(p, W1a_ref[...], preferred_element_type=jnp.float32)
          + jnp.dot(z_ref[0], W1b_ref[...], preferred_element_type=jnp.float32)
          + b1_ref[...])
    m = jnp.mean(h1, axis=-1, keepdims=True)
    v = jnp.mean((h1 - m) ** 2, axis=-1, keepdims=True)
    h1 = (h1 - m) / jnp.sqrt(v + 1e-5) * g_ref[...] + bl_ref[...]
    h1 = jnp.maximum(h1, 0.0)
    h = jnp.dot(h1, W2_ref[...], preferred_element_type=jnp.float32) + b2_ref[...]
    _vq_tail(h, E_ref[0], e_ref, idx_ref, loss_ref, r)


def _root_call():
    return pl.pallas_call(
        _root_body,
        grid=(NRB,),
        in_specs=[
            pl.BlockSpec((1, RB, C), lambda r: (0, r, 0)),
            pl.BlockSpec((1, N_E, C), lambda r: (0, 0, 0)),
        ],
        out_specs=(
            pl.BlockSpec((RB, 1, 1, C), lambda r: (r, 0, 0, 0)),
            pl.BlockSpec((1, 1, RB), lambda r: (r, 0, 0)),
            pl.BlockSpec((1, 1, C), lambda r: (0, 0, 0)),
        ),
        out_shape=(
            jax.ShapeDtypeStruct((R, V, 1, C), jnp.float32),
            jax.ShapeDtypeStruct((NRB, 1, RB), jnp.int32),
            jax.ShapeDtypeStruct((1, 1, C), jnp.float32),
        ),
    )


@functools.lru_cache(maxsize=None)
def _wave_call(nj):
    full = lambda j, r, jids, pids: (0, 0)
    return pl.pallas_call(
        _wave_body,
        grid_spec=pltpu.PrefetchScalarGridSpec(
            num_scalar_prefetch=2,
            grid=(nj, NRB),
            in_specs=[
                pl.BlockSpec((RB, 1, 1, C),
                             lambda j, r, jids, pids: (r, pids[j], 0, 0)),
                pl.BlockSpec((1, RB, C),
                             lambda j, r, jids, pids: (jids[j], r, 0)),
                pl.BlockSpec((1, N_E, C),
                             lambda j, r, jids, pids: (jids[j], 0, 0)),
                pl.BlockSpec((C, HID), full),
                pl.BlockSpec((C, HID), full),
                pl.BlockSpec((1, HID), full),
                pl.BlockSpec((1, HID), full),
                pl.BlockSpec((1, HID), full),
                pl.BlockSpec((HID, C), full),
                pl.BlockSpec((1, C), full),
            ],
            out_specs=(
                pl.BlockSpec((RB, 1, 1, C),
                             lambda j, r, jids, pids: (r, jids[j], 0, 0)),
                pl.BlockSpec((1, 1, RB),
                             lambda j, r, jids, pids: (j * NRB + r, 0, 0)),
                pl.BlockSpec((1, 1, C),
                             lambda j, r, jids, pids: (j, 0, 0)),
            ),
        ),
        out_shape=(
            jax.ShapeDtypeStruct((R, V, 1, C), jnp.float32),
            jax.ShapeDtypeStruct((nj * NRB, 1, RB), jnp.int32),
            jax.ShapeDtypeStruct((nj, 1, C), jnp.float32),
        ),
        input_output_aliases={2: 0},
    )


def kernel(z, emb, W1, b1, g_ln, b_ln, W2, b2):
    zt = jnp.transpose(z, (2, 0, 1, 3)).reshape(V, R, C)
    W1a, W1b = W1[:C], W1[C:]
    b1r = b1.reshape(1, HID)
    gr = g_ln.reshape(1, HID)
    blr = b_ln.reshape(1, HID)
    b2r = b2.reshape(1, C)

    i_all = [None] * V
    ebuf, idx, lp = _root_call()(zt, emb)
    i_all[0] = idx.reshape(R)
    loss_sum = jnp.sum(lp)
    for wave in WAVES[1:]:
        nj = len(wave)
        jids = jnp.array([j for j, _ in wave], jnp.int32)
        pids = jnp.array([p for _, p in wave], jnp.int32)
        ebuf, idx, lp = _wave_call(nj)(jids, pids, ebuf, zt, emb, W1a, W1b,
                                       b1r, gr, blr, W2, b2r)
        idx = idx.reshape(nj, R)
        for k, (j, _) in enumerate(wave):
            i_all[j] = idx[k]
        loss_sum = loss_sum + jnp.sum(lp)

    z_q = ebuf.reshape(B, T, V, C)
    indices = jnp.stack(i_all, axis=0).reshape(V, B, T).transpose(1, 2, 0)
    total = (1.0 + BETA) * loss_sum / (V * R * C)
    return z_q, total, indices


# speedup vs baseline: 1.8983x; 1.0810x over previous
import functools

import jax
import jax.numpy as jnp
from jax.experimental import pallas as pl
from jax.experimental.pallas import tpu as pltpu

B, T, V, C = 32, 256, 32, 128
N_E = 1024
HID = 256
BETA = 0.25
R = B * T
RB = 512
NRB = R // RB

WAVES = (
    ((0, 0),),
    ((1, 0), (6, 0), (11, 0)),
    ((2, 1), (7, 6), (12, 11)),
    ((3, 2), (8, 7), (13, 12), (16, 12), (24, 12)),
    ((4, 3), (9, 8), (14, 13), (17, 16), (25, 24)),
    ((5, 4), (10, 9), (15, 14), (18, 17), (26, 25)),
    ((19, 18), (27, 26)),
    ((20, 19), (28, 27)),
    ((21, 20), (29, 28)),
    ((22, 21), (30, 29)),
    ((23, 22), (31, 30)),
)


def _vq_tail(h, E, e_ref, idx_ref, loss_ref, en_ref, r):
    @pl.when(r == 0)
    def _():
        en_ref[...] = jnp.sum(E * E, axis=1)[None, :]

    d2 = en_ref[...] - 2.0 * jnp.dot(h, E.T, preferred_element_type=jnp.float32)
    idx = jnp.argmin(d2, axis=1).astype(jnp.int32)
    oh = (jax.lax.broadcasted_iota(jnp.int32, (RB, N_E), 1) == idx[:, None])
    e = jnp.dot(oh.astype(jnp.float32), E, preferred_element_type=jnp.float32)
    diff = e - h
    e_ref[:, 0, 0, :] = e
    idx_ref[0, 0] = idx
    part = jnp.sum(diff * diff, axis=0, keepdims=True)[None]

    @pl.when(r == 0)
    def _():
        loss_ref[...] = part

    @pl.when(r != 0)
    def _():
        loss_ref[...] += part


def _root_body(z_ref, E_ref, e_ref, idx_ref, loss_ref, en_ref):
    r = pl.program_id(0)
    _vq_tail(z_ref[:, 0, 0, :], E_ref[0], e_ref, idx_ref, loss_ref, en_ref, r)


def _wave_body(jids_ref, pids_ref, ebuf_ref, z_ref, E_ref, W1a_ref, W1b_ref,
               b1_ref, g_ref, bl_ref, W2_ref, b2_ref, e_ref, idx_ref,
               loss_ref, en_ref):
    r = pl.program_id(1)
    p = ebuf_ref[:, 0, 0, :]
    h1 = (jnp.dot(p, W1a_ref[...], preferred_element_type=jnp.float32)
          + jnp.dot(z_ref[:, 0, 0, :], W1b_ref[...],
                    preferred_element_type=jnp.float32)
          + b1_ref[...])
    m = jnp.mean(h1, axis=-1, keepdims=True)
    v = jnp.mean((h1 - m) ** 2, axis=-1, keepdims=True)
    h1 = (h1 - m) / jnp.sqrt(v + 1e-5) * g_ref[...] + bl_ref[...]
    h1 = jnp.maximum(h1, 0.0)
    h = jnp.dot(h1, W2_ref[...], preferred_element_type=jnp.float32) + b2_ref[...]
    _vq_tail(h, E_ref[0], e_ref, idx_ref, loss_ref, en_ref, r)


def _root_call():
    return pl.pallas_call(
        _root_body,
        grid=(NRB,),
        in_specs=[
            pl.BlockSpec((RB, 1, 1, C), lambda r: (r, 0, 0, 0)),
            pl.BlockSpec((1, N_E, C), lambda r: (0, 0, 0)),
        ],
        out_specs=(
            pl.BlockSpec((RB, 1, 1, C), lambda r: (r, 0, 0, 0)),
            pl.BlockSpec((1, 1, RB), lambda r: (r, 0, 0)),
            pl.BlockSpec((1, 1, C), lambda r: (0, 0, 0)),
        ),
        out_shape=(
            jax.ShapeDtypeStruct((R, V, 1, C), jnp.float32),
            jax.ShapeDtypeStruct((NRB, 1, RB), jnp.int32),
            jax.ShapeDtypeStruct((1, 1, C), jnp.float32),
        ),
        scratch_shapes=[pltpu.VMEM((1, N_E), jnp.float32)],
    )


@functools.lru_cache(maxsize=None)
def _wave_call(nj):
    full = lambda j, r, jids, pids: (0, 0)
    return pl.pallas_call(
        _wave_body,
        grid_spec=pltpu.PrefetchScalarGridSpec(
            num_scalar_prefetch=2,
            grid=(nj, NRB),
            in_specs=[
                pl.BlockSpec((RB, 1, 1, C),
                             lambda j, r, jids, pids: (r, pids[j], 0, 0)),
                pl.BlockSpec((RB, 1, 1, C),
                             lambda j, r, jids, pids: (r, jids[j], 0, 0)),
                pl.BlockSpec((1, N_E, C),
                             lambda j, r, jids, pids: (jids[j], 0, 0)),
                pl.BlockSpec((C, HID), full),
                pl.BlockSpec((C, HID), full),
                pl.BlockSpec((1, HID), full),
                pl.BlockSpec((1, HID), full),
                pl.BlockSpec((1, HID), full),
                pl.BlockSpec((HID, C), full),
                pl.BlockSpec((1, C), full),
            ],
            out_specs=(
                pl.BlockSpec((RB, 1, 1, C),
                             lambda j, r, jids, pids: (r, jids[j], 0, 0)),
                pl.BlockSpec((1, 1, RB),
                             lambda j, r, jids, pids: (j * NRB + r, 0, 0)),
                pl.BlockSpec((1, 1, C),
                             lambda j, r, jids, pids: (j, 0, 0)),
            ),
            scratch_shapes=[pltpu.VMEM((1, N_E), jnp.float32)],
        ),
        out_shape=(
            jax.ShapeDtypeStruct((R, V, 1, C), jnp.float32),
            jax.ShapeDtypeStruct((nj * NRB, 1, RB), jnp.int32),
            jax.ShapeDtypeStruct((nj, 1, C), jnp.float32),
        ),
        input_output_aliases={2: 0},
    )


def kernel(z, emb, W1, b1, g_ln, b_ln, W2, b2):
    zt = z.reshape(R, V, 1, C)
    W1a, W1b = W1[:C], W1[C:]
    b1r = b1.reshape(1, HID)
    gr = g_ln.reshape(1, HID)
    blr = b_ln.reshape(1, HID)
    b2r = b2.reshape(1, C)

    i_all = [None] * V
    ebuf, idx, lp = _root_call()(zt, emb)
    i_all[0] = idx.reshape(R)
    loss_sum = jnp.sum(lp)
    for wave in WAVES[1:]:
        nj = len(wave)
        jids = jnp.array([j for j, _ in wave], jnp.int32)
        pids = jnp.array([p for _, p in wave], jnp.int32)
        ebuf, idx, lp = _wave_call(nj)(jids, pids, ebuf, zt, emb, W1a, W1b,
                                       b1r, gr, blr, W2, b2r)
        idx = idx.reshape(nj, R)
        for k, (j, _) in enumerate(wave):
            i_all[j] = idx[k]
        loss_sum = loss_sum + jnp.sum(lp)

    z_q = ebuf.reshape(B, T, V, C)
    indices = jnp.stack(i_all, axis=0).reshape(V, B, T).transpose(1, 2, 0)
    total = (1.0 + BETA) * loss_sum / (V * R * C)
    return z_q, total, indices


# speedup vs baseline: 2.3037x; 1.2135x over previous
import functools

import jax
import jax.numpy as jnp
from jax.experimental import pallas as pl
from jax.experimental.pallas import tpu as pltpu

B, T, V, C = 32, 256, 32, 128
N_E = 1024
HID = 256
BETA = 0.25
R = B * T
RB = 1024
NRB = R // RB

WAVES = (
    ((0, 0),),
    ((1, 0), (6, 0), (11, 0)),
    ((2, 1), (7, 6), (12, 11)),
    ((3, 2), (8, 7), (13, 12), (16, 12), (24, 12)),
    ((4, 3), (9, 8), (14, 13), (17, 16), (25, 24)),
    ((5, 4), (10, 9), (15, 14), (18, 17), (26, 25)),
    ((19, 18), (27, 26)),
    ((20, 19), (28, 27)),
    ((21, 20), (29, 28)),
    ((22, 21), (30, 29)),
    ((23, 22), (31, 30)),
)


def _vq_tail(h, E, e_ref, idx_ref, loss_ref, en_ref, r):
    @pl.when(r == 0)
    def _():
        en_ref[...] = jnp.sum(E * E, axis=1)[None, :]

    d2 = en_ref[...] - 2.0 * jnp.dot(h, E.T, preferred_element_type=jnp.float32)
    idx = jnp.argmin(d2, axis=1).astype(jnp.int32)
    oh = (jax.lax.broadcasted_iota(jnp.int32, (RB, N_E), 1) == idx[:, None])
    e = jnp.dot(oh.astype(jnp.float32), E, preferred_element_type=jnp.float32)
    diff = e - h
    e_ref[:, 0, 0, :] = e
    idx_ref[0, 0] = idx
    part = jnp.sum(diff * diff, axis=0, keepdims=True)[None]

    @pl.when(r == 0)
    def _():
        loss_ref[...] = part

    @pl.when(r != 0)
    def _():
        loss_ref[...] += part


def _root_body(z_ref, E_ref, e_ref, idx_ref, loss_ref, en_ref):
    r = pl.program_id(0)
    _vq_tail(z_ref[:, 0, 0, :], E_ref[0], e_ref, idx_ref, loss_ref, en_ref, r)


def _wave_body(jids_ref, pids_ref, ebuf_ref, z_ref, E_ref, W1a_ref, W1b_ref,
               b1_ref, g_ref, bl_ref, W2_ref, b2_ref, e_ref, idx_ref,
               loss_ref, en_ref):
    r = pl.program_id(1)
    p = ebuf_ref[:, 0, 0, :]
    h1 = (jnp.dot(p, W1a_ref[...], preferred_element_type=jnp.float32)
          + jnp.dot(z_ref[:, 0, 0, :], W1b_ref[...],
                    preferred_element_type=jnp.float32)
          + b1_ref[...])
    m = jnp.mean(h1, axis=-1, keepdims=True)
    v = jnp.mean((h1 - m) ** 2, axis=-1, keepdims=True)
    h1 = (h1 - m) / jnp.sqrt(v + 1e-5) * g_ref[...] + bl_ref[...]
    h1 = jnp.maximum(h1, 0.0)
    h = jnp.dot(h1, W2_ref[...], preferred_element_type=jnp.float32) + b2_ref[...]
    _vq_tail(h, E_ref[0], e_ref, idx_ref, loss_ref, en_ref, r)


def _root_call():
    return pl.pallas_call(
        _root_body,
        grid=(NRB,),
        in_specs=[
            pl.BlockSpec((RB, 1, 1, C), lambda r: (r, 0, 0, 0)),
            pl.BlockSpec((1, N_E, C), lambda r: (0, 0, 0)),
        ],
        out_specs=(
            pl.BlockSpec((RB, 1, 1, C), lambda r: (r, 0, 0, 0)),
            pl.BlockSpec((1, 1, RB), lambda r: (r, 0, 0)),
            pl.BlockSpec((1, 1, C), lambda r: (0, 0, 0)),
        ),
        out_shape=(
            jax.ShapeDtypeStruct((R, V, 1, C), jnp.float32),
            jax.ShapeDtypeStruct((NRB, 1, RB), jnp.int32),
            jax.ShapeDtypeStruct((1, 1, C), jnp.float32),
        ),
        scratch_shapes=[pltpu.VMEM((1, N_E), jnp.float32)],
    )


@functools.lru_cache(maxsize=None)
def _wave_call(nj):
    full = lambda j, r, jids, pids: (0, 0)
    return pl.pallas_call(
        _wave_body,
        grid_spec=pltpu.PrefetchScalarGridSpec(
            num_scalar_prefetch=2,
            grid=(nj, NRB),
            in_specs=[
                pl.BlockSpec((RB, 1, 1, C),
                             lambda j, r, jids, pids: (r, pids[j], 0, 0)),
                pl.BlockSpec((RB, 1, 1, C),
                             lambda j, r, jids, pids: (r, jids[j], 0, 0)),
                pl.BlockSpec((1, N_E, C),
                             lambda j, r, jids, pids: (jids[j], 0, 0)),
                pl.BlockSpec((C, HID), full),
                pl.BlockSpec((C, HID), full),
                pl.BlockSpec((1, HID), full),
                pl.BlockSpec((1, HID), full),
                pl.BlockSpec((1, HID), full),
                pl.BlockSpec((HID, C), full),
                pl.BlockSpec((1, C), full),
            ],
            out_specs=(
                pl.BlockSpec((RB, 1, 1, C),
                             lambda j, r, jids, pids: (r, jids[j], 0, 0)),
                pl.BlockSpec((1, 1, RB),
                             lambda j, r, jids, pids: (j * NRB + r, 0, 0)),
                pl.BlockSpec((1, 1, C),
                             lambda j, r, jids, pids: (j, 0, 0)),
            ),
            scratch_shapes=[pltpu.VMEM((1, N_E), jnp.float32)],
        ),
        out_shape=(
            jax.ShapeDtypeStruct((R, V, 1, C), jnp.float32),
            jax.ShapeDtypeStruct((nj * NRB, 1, RB), jnp.int32),
            jax.ShapeDtypeStruct((nj, 1, C), jnp.float32),
        ),
        input_output_aliases={2: 0},
    )


def kernel(z, emb, W1, b1, g_ln, b_ln, W2, b2):
    zt = z.reshape(R, V, 1, C)
    W1a, W1b = W1[:C], W1[C:]
    b1r = b1.reshape(1, HID)
    gr = g_ln.reshape(1, HID)
    blr = b_ln.reshape(1, HID)
    b2r = b2.reshape(1, C)

    i_all = [None] * V
    ebuf, idx, lp = _root_call()(zt, emb)
    i_all[0] = idx.reshape(R)
    loss_sum = jnp.sum(lp)
    for wave in WAVES[1:]:
        nj = len(wave)
        jids = jnp.array([j for j, _ in wave], jnp.int32)
        pids = jnp.array([p for _, p in wave], jnp.int32)
        ebuf, idx, lp = _wave_call(nj)(jids, pids, ebuf, zt, emb, W1a, W1b,
                                       b1r, gr, blr, W2, b2r)
        idx = idx.reshape(nj, R)
        for k, (j, _) in enumerate(wave):
            i_all[j] = idx[k]
        loss_sum = loss_sum + jnp.sum(lp)

    z_q = ebuf.reshape(B, T, V, C)
    indices = jnp.stack(i_all, axis=0).reshape(V, B, T).transpose(1, 2, 0)
    total = (1.0 + BETA) * loss_sum / (V * R * C)
    return z_q, total, indices


# speedup vs baseline: 2.5496x; 1.1068x over previous
import functools

import jax
import jax.numpy as jnp
from jax.experimental import pallas as pl
from jax.experimental.pallas import tpu as pltpu

B, T, V, C = 32, 256, 32, 128
N_E = 1024
HID = 256
BETA = 0.25
R = B * T
RB = 2048
NRB = R // RB

WAVES = (
    ((0, 0),),
    ((1, 0), (6, 0), (11, 0)),
    ((2, 1), (7, 6), (12, 11)),
    ((3, 2), (8, 7), (13, 12), (16, 12), (24, 12)),
    ((4, 3), (9, 8), (14, 13), (17, 16), (25, 24)),
    ((5, 4), (10, 9), (15, 14), (18, 17), (26, 25)),
    ((19, 18), (27, 26)),
    ((20, 19), (28, 27)),
    ((21, 20), (29, 28)),
    ((22, 21), (30, 29)),
    ((23, 22), (31, 30)),
)


def _vq_tail(h, E, e_ref, idx_ref, loss_ref, en_ref, r):
    @pl.when(r == 0)
    def _():
        en_ref[...] = jnp.sum(E * E, axis=1)[None, :]

    d2 = en_ref[...] - 2.0 * jnp.dot(h, E.T, preferred_element_type=jnp.float32)
    idx = jnp.argmin(d2, axis=1).astype(jnp.int32)
    oh = (jax.lax.broadcasted_iota(jnp.int32, (RB, N_E), 1) == idx[:, None])
    e = jnp.dot(oh.astype(jnp.float32), E, preferred_element_type=jnp.float32)
    diff = e - h
    e_ref[:, 0, 0, :] = e
    idx_ref[0, 0] = idx
    part = jnp.sum(diff * diff, axis=0, keepdims=True)[None]

    @pl.when(r == 0)
    def _():
        loss_ref[...] = part

    @pl.when(r != 0)
    def _():
        loss_ref[...] += part


def _root_body(z_ref, E_ref, e_ref, idx_ref, loss_ref, en_ref):
    r = pl.program_id(0)
    _vq_tail(z_ref[:, 0, 0, :], E_ref[0], e_ref, idx_ref, loss_ref, en_ref, r)


def _wave_body(jids_ref, pids_ref, ebuf_ref, z_ref, E_ref, W1a_ref, W1b_ref,
               b1_ref, g_ref, bl_ref, W2_ref, b2_ref, e_ref, idx_ref,
               loss_ref, en_ref):
    r = pl.program_id(1)
    p = ebuf_ref[:, 0, 0, :]
    h1 = (jnp.dot(p, W1a_ref[...], preferred_element_type=jnp.float32)
          + jnp.dot(z_ref[:, 0, 0, :], W1b_ref[...],
                    preferred_element_type=jnp.float32)
          + b1_ref[...])
    m = jnp.mean(h1, axis=-1, keepdims=True)
    v = jnp.mean((h1 - m) ** 2, axis=-1, keepdims=True)
    h1 = (h1 - m) / jnp.sqrt(v + 1e-5) * g_ref[...] + bl_ref[...]
    h1 = jnp.maximum(h1, 0.0)
    h = jnp.dot(h1, W2_ref[...], preferred_element_type=jnp.float32) + b2_ref[...]
    _vq_tail(h, E_ref[0], e_ref, idx_ref, loss_ref, en_ref, r)


def _root_call():
    return pl.pallas_call(
        _root_body,
        grid=(NRB,),
        in_specs=[
            pl.BlockSpec((RB, 1, 1, C), lambda r: (r, 0, 0, 0)),
            pl.BlockSpec((1, N_E, C), lambda r: (0, 0, 0)),
        ],
        out_specs=(
            pl.BlockSpec((RB, 1, 1, C), lambda r: (r, 0, 0, 0)),
            pl.BlockSpec((1, 1, RB), lambda r: (r, 0, 0)),
            pl.BlockSpec((1, 1, C), lambda r: (0, 0, 0)),
        ),
        out_shape=(
            jax.ShapeDtypeStruct((R, V, 1, C), jnp.float32),
            jax.ShapeDtypeStruct((NRB, 1, RB), jnp.int32),
            jax.ShapeDtypeStruct((1, 1, C), jnp.float32),
        ),
        scratch_shapes=[pltpu.VMEM((1, N_E), jnp.float32)],
    )


@functools.lru_cache(maxsize=None)
def _wave_call(nj):
    full = lambda j, r, jids, pids: (0, 0)
    return pl.pallas_call(
        _wave_body,
        grid_spec=pltpu.PrefetchScalarGridSpec(
            num_scalar_prefetch=2,
            grid=(nj, NRB),
            in_specs=[
                pl.BlockSpec((RB, 1, 1, C),
                             lambda j, r, jids, pids: (r, pids[j], 0, 0)),
                pl.BlockSpec((RB, 1, 1, C),
                             lambda j, r, jids, pids: (r, jids[j], 0, 0)),
                pl.BlockSpec((1, N_E, C),
                             lambda j, r, jids, pids: (jids[j], 0, 0)),
                pl.BlockSpec((C, HID), full),
                pl.BlockSpec((C, HID), full),
                pl.BlockSpec((1, HID), full),
                pl.BlockSpec((1, HID), full),
                pl.BlockSpec((1, HID), full),
                pl.BlockSpec((HID, C), full),
                pl.BlockSpec((1, C), full),
            ],
            out_specs=(
                pl.BlockSpec((RB, 1, 1, C),
                             lambda j, r, jids, pids: (r, jids[j], 0, 0)),
                pl.BlockSpec((1, 1, RB),
                             lambda j, r, jids, pids: (j * NRB + r, 0, 0)),
                pl.BlockSpec((1, 1, C),
                             lambda j, r, jids, pids: (j, 0, 0)),
            ),
            scratch_shapes=[pltpu.VMEM((1, N_E), jnp.float32)],
        ),
        out_shape=(
            jax.ShapeDtypeStruct((R, V, 1, C), jnp.float32),
            jax.ShapeDtypeStruct((nj * NRB, 1, RB), jnp.int32),
            jax.ShapeDtypeStruct((nj, 1, C), jnp.float32),
        ),
        input_output_aliases={2: 0},
    )


def kernel(z, emb, W1, b1, g_ln, b_ln, W2, b2):
    zt = z.reshape(R, V, 1, C)
    W1a, W1b = W1[:C], W1[C:]
    b1r = b1.reshape(1, HID)
    gr = g_ln.reshape(1, HID)
    blr = b_ln.reshape(1, HID)
    b2r = b2.reshape(1, C)

    i_all = [None] * V
    ebuf, idx, lp = _root_call()(zt, emb)
    i_all[0] = idx.reshape(R)
    loss_sum = jnp.sum(lp)
    for wave in WAVES[1:]:
        nj = len(wave)
        jids = jnp.array([j for j, _ in wave], jnp.int32)
        pids = jnp.array([p for _, p in wave], jnp.int32)
        ebuf, idx, lp = _wave_call(nj)(jids, pids, ebuf, zt, emb, W1a, W1b,
                                       b1r, gr, blr, W2, b2r)
        idx = idx.reshape(nj, R)
        for k, (j, _) in enumerate(wave):
            i_all[j] = idx[k]
        loss_sum = loss_sum + jnp.sum(lp)

    z_q = ebuf.reshape(B, T, V, C)
    indices = jnp.stack(i_all, axis=0).reshape(V, B, T).transpose(1, 2, 0)
    total = (1.0 + BETA) * loss_sum / (V * R * C)
    return z_q, total, indices


# speedup vs baseline: 2.8154x; 1.1043x over previous
import functools

import jax
import jax.numpy as jnp
from jax.experimental import pallas as pl
from jax.experimental.pallas import tpu as pltpu

B, T, V, C = 32, 256, 32, 128
N_E = 1024
HID = 256
BETA = 0.25
R = B * T
RB = 4096
NRB = R // RB

WAVES = (
    ((0, 0),),
    ((1, 0), (6, 0), (11, 0)),
    ((2, 1), (7, 6), (12, 11)),
    ((3, 2), (8, 7), (13, 12), (16, 12), (24, 12)),
    ((4, 3), (9, 8), (14, 13), (17, 16), (25, 24)),
    ((5, 4), (10, 9), (15, 14), (18, 17), (26, 25)),
    ((19, 18), (27, 26)),
    ((20, 19), (28, 27)),
    ((21, 20), (29, 28)),
    ((22, 21), (30, 29)),
    ((23, 22), (31, 30)),
)


def _vq_tail(h, E, e_ref, idx_ref, loss_ref, en_ref, r):
    @pl.when(r == 0)
    def _():
        en_ref[...] = jnp.sum(E * E, axis=1)[None, :]

    d2 = en_ref[...] - 2.0 * jnp.dot(h, E.T, preferred_element_type=jnp.float32)
    idx = jnp.argmin(d2, axis=1).astype(jnp.int32)
    oh = (jax.lax.broadcasted_iota(jnp.int32, (RB, N_E), 1) == idx[:, None])
    e = jnp.dot(oh.astype(jnp.float32), E, preferred_element_type=jnp.float32)
    diff = e - h
    e_ref[:, 0, 0, :] = e
    idx_ref[0, 0] = idx
    part = jnp.sum(diff * diff, axis=0, keepdims=True)[None]

    @pl.when(r == 0)
    def _():
        loss_ref[...] = part

    @pl.when(r != 0)
    def _():
        loss_ref[...] += part


def _root_body(z_ref, E_ref, e_ref, idx_ref, loss_ref, en_ref):
    r = pl.program_id(0)
    _vq_tail(z_ref[:, 0, 0, :], E_ref[0], e_ref, idx_ref, loss_ref, en_ref, r)


def _wave_body(jids_ref, pids_ref, ebuf_ref, z_ref, E_ref, W1a_ref, W1b_ref,
               b1_ref, g_ref, bl_ref, W2_ref, b2_ref, e_ref, idx_ref,
               loss_ref, en_ref):
    r = pl.program_id(1)
    p = ebuf_ref[:, 0, 0, :]
    h1 = (jnp.dot(p, W1a_ref[...], preferred_element_type=jnp.float32)
          + jnp.dot(z_ref[:, 0, 0, :], W1b_ref[...],
                    preferred_element_type=jnp.float32)
          + b1_ref[...])
    m = jnp.mean(h1, axis=-1, keepdims=True)
    v = jnp.mean((h1 - m) ** 2, axis=-1, keepdims=True)
    h1 = (h1 - m) / jnp.sqrt(v + 1e-5) * g_ref[...] + bl_ref[...]
    h1 = jnp.maximum(h1, 0.0)
    h = jnp.dot(h1, W2_ref[...], preferred_element_type=jnp.float32) + b2_ref[...]
    _vq_tail(h, E_ref[0], e_ref, idx_ref, loss_ref, en_ref, r)


def _root_call():
    return pl.pallas_call(
        _root_body,
        grid=(NRB,),
        in_specs=[
            pl.BlockSpec((RB, 1, 1, C), lambda r: (r, 0, 0, 0)),
            pl.BlockSpec((1, N_E, C), lambda r: (0, 0, 0)),
        ],
        out_specs=(
            pl.BlockSpec((RB, 1, 1, C), lambda r: (r, 0, 0, 0)),
            pl.BlockSpec((1, 1, RB), lambda r: (r, 0, 0)),
            pl.BlockSpec((1, 1, C), lambda r: (0, 0, 0)),
        ),
        out_shape=(
            jax.ShapeDtypeStruct((R, V, 1, C), jnp.float32),
            jax.ShapeDtypeStruct((NRB, 1, RB), jnp.int32),
            jax.ShapeDtypeStruct((1, 1, C), jnp.float32),
        ),
        scratch_shapes=[pltpu.VMEM((1, N_E), jnp.float32)],
    )


@functools.lru_cache(maxsize=None)
def _wave_call(nj):
    full = lambda j, r, jids, pids: (0, 0)
    return pl.pallas_call(
        _wave_body,
        grid_spec=pltpu.PrefetchScalarGridSpec(
            num_scalar_prefetch=2,
            grid=(nj, NRB),
            in_specs=[
                pl.BlockSpec((RB, 1, 1, C),
                             lambda j, r, jids, pids: (r, pids[j], 0, 0)),
                pl.BlockSpec((RB, 1, 1, C),
                             lambda j, r, jids, pids: (r, jids[j], 0, 0)),
                pl.BlockSpec((1, N_E, C),
                             lambda j, r, jids, pids: (jids[j], 0, 0)),
                pl.BlockSpec((C, HID), full),
                pl.BlockSpec((C, HID), full),
                pl.BlockSpec((1, HID), full),
                pl.BlockSpec((1, HID), full),
                pl.BlockSpec((1, HID), full),
                pl.BlockSpec((HID, C), full),
                pl.BlockSpec((1, C), full),
            ],
            out_specs=(
                pl.BlockSpec((RB, 1, 1, C),
                             lambda j, r, jids, pids: (r, jids[j], 0, 0)),
                pl.BlockSpec((1, 1, RB),
                             lambda j, r, jids, pids: (j * NRB + r, 0, 0)),
                pl.BlockSpec((1, 1, C),
                             lambda j, r, jids, pids: (j, 0, 0)),
            ),
            scratch_shapes=[pltpu.VMEM((1, N_E), jnp.float32)],
        ),
        out_shape=(
            jax.ShapeDtypeStruct((R, V, 1, C), jnp.float32),
            jax.ShapeDtypeStruct((nj * NRB, 1, RB), jnp.int32),
            jax.ShapeDtypeStruct((nj, 1, C), jnp.float32),
        ),
        input_output_aliases={2: 0},
    )


def kernel(z, emb, W1, b1, g_ln, b_ln, W2, b2):
    zt = z.reshape(R, V, 1, C)
    W1a, W1b = W1[:C], W1[C:]
    b1r = b1.reshape(1, HID)
    gr = g_ln.reshape(1, HID)
    blr = b_ln.reshape(1, HID)
    b2r = b2.reshape(1, C)

    i_all = [None] * V
    ebuf, idx, lp = _root_call()(zt, emb)
    i_all[0] = idx.reshape(R)
    loss_sum = jnp.sum(lp)
    for wave in WAVES[1:]:
        nj = len(wave)
        jids = jnp.array([j for j, _ in wave], jnp.int32)
        pids = jnp.array([p for _, p in wave], jnp.int32)
        ebuf, idx, lp = _wave_call(nj)(jids, pids, ebuf, zt, emb, W1a, W1b,
                                       b1r, gr, blr, W2, b2r)
        idx = idx.reshape(nj, R)
        for k, (j, _) in enumerate(wave):
            i_all[j] = idx[k]
        loss_sum = loss_sum + jnp.sum(lp)

    z_q = ebuf.reshape(B, T, V, C)
    indices = jnp.stack(i_all, axis=0).reshape(V, B, T).transpose(1, 2, 0)
    total = (1.0 + BETA) * loss_sum / (V * R * C)
    return z_q, total, indices
